# Initial kernel scaffold; baseline (speedup 1.0000x reference)
import jax
import jax.numpy as jnp
from jax.experimental import pallas as pl


def kernel(inputs, timestamp, train, W, b):
    raise NotImplementedError("write your pallas kernel here")


# speedup vs baseline: 1.4278x; 1.4278x over previous
import jax
import jax.numpy as jnp
from jax import lax
from jax.experimental import pallas as pl
from jax.experimental.pallas import tpu as pltpu
from jax.experimental.pallas import tpu_sc as plsc

_B, _L = 4096, 200
_NBINS = 101
_D = 64
_NC, _NS = 2, 16
_NW = _NC * _NS
_RW = _B // _NW
_PW = _RW * _L
_G = 128
_NG = _PW // _G

_OFFS = tuple(range(0, _L - 16, 16)) + (_L - 16,)


def _sc_body(ts_hbm, table_hbm, out_hbm, ts_v, idx_v, rows_v, sem0, sem1):
    wid = lax.axis_index("s") * _NC + lax.axis_index("c")
    row0 = wid * _RW
    pltpu.sync_copy(ts_hbm.at[pl.ds(row0, _RW)], ts_v)

    def row_body(r, carry):
        p0 = r * _L
        for off in _OFFS:
            a = ts_v[r, pl.ds(off + 1, 16)]
            c = ts_v[r, pl.ds(off, 16)]
            v = (a - c) * jnp.float32(_NBINS - 1)
            i = jnp.minimum(jnp.maximum(v.astype(jnp.int32), 0), _NBINS - 1)
            idx_v[pl.ds(p0 + off, 16)] = i
        return carry

    lax.fori_loop(0, _RW, row_body, 0)

    out0 = wid * _PW

    def g_body(p, carry):
        j = 2 * p
        c0 = pltpu.async_copy(
            table_hbm.at[idx_v.at[pl.ds(j * _G, _G)]], rows_v.at[0], sem0)
        c1 = pltpu.async_copy(
            table_hbm.at[idx_v.at[pl.ds((j + 1) * _G, _G)]], rows_v.at[1], sem1)
        c0.wait()
        pltpu.sync_copy(rows_v.at[0], out_hbm.at[pl.ds(out0 + j * _G, _G)])
        c1.wait()
        pltpu.sync_copy(rows_v.at[1], out_hbm.at[pl.ds(out0 + (j + 1) * _G, _G)])
        return carry

    lax.fori_loop(0, _NG // 2, g_body, 0)


def kernel(inputs, timestamp, train, W, b):
    del inputs, train
    table = W.T + b[None, :]
    mesh = plsc.VectorSubcoreMesh(core_axis_name="c", subcore_axis_name="s")
    k = pl.kernel(
        _sc_body,
        out_type=jax.ShapeDtypeStruct((_B * _L, _D), jnp.float32),
        mesh=mesh,
        compiler_params=pltpu.CompilerParams(use_tc_tiling_on_sc=False),
        scratch_types=[
            pltpu.VMEM((_RW, _L + 1), jnp.float32),
            pltpu.VMEM((_PW,), jnp.int32),
            pltpu.VMEM((2, _G, _D), jnp.float32),
            pltpu.SemaphoreType.DMA,
            pltpu.SemaphoreType.DMA,
        ],
    )
    out = k(timestamp, table)
    return (out.reshape(_B, _L, _D), timestamp[:, :-1])


# speedup vs baseline: 13.6947x; 9.5918x over previous
import jax
import jax.numpy as jnp
from jax import lax
from jax.experimental import pallas as pl
from jax.experimental.pallas import tpu as pltpu
from jax.experimental.pallas import tpu_sc as plsc

_B, _L = 4096, 200
_NBINS = 101
_D = 64
_NC, _NS = 2, 16
_NW = _NC * _NS
_RW = _B // _NW
_PW = _RW * _L
_CH = 512
_NCH = _PW // _CH

_OFFS = tuple(range(0, _L - 16, 16)) + (_L - 16,)


def _sc_body(ts_hbm, table_hbm, out_hbm, ts_v, idx_v, tab_v, stage_v, sem0, sem1):
    wid = lax.axis_index("s") * _NC + lax.axis_index("c")
    row0 = wid * _RW
    pltpu.sync_copy(table_hbm, tab_v)
    pltpu.sync_copy(ts_hbm.at[pl.ds(row0, _RW)], ts_v)

    def row_body(r, carry):
        p0 = r * _L
        for off in _OFFS:
            a = ts_v[r, pl.ds(off + 1, 16)]
            c = ts_v[r, pl.ds(off, 16)]
            v = (a - c) * jnp.float32(_NBINS - 1)
            i = jnp.minimum(jnp.maximum(v.astype(jnp.int32), 0), _NBINS - 1)
            idx_v[pl.ds(p0 + off, 16)] = i * _D
        return carry

    lax.fori_loop(0, _RW, row_body, 0)

    out0 = wid * _PW * _D

    def fill(buf, ch):
        def grp(g, carry):
            iv = idx_v[pl.ds(ch * _CH + g * 16, 16)]
            sb = g * (16 * _D)
            for r in range(16):
                base = iv[r]
                for c in range(0, _D, 16):
                    stage_v[buf, pl.ds(sb + r * _D + c, 16)] = tab_v[pl.ds(base + c, 16)]
            return carry
        lax.fori_loop(0, _CH // 16, grp, 0)

    def pair_body(p, carry):
        ch0 = 2 * p
        fill(0, ch0)
        c0 = pltpu.async_copy(
            stage_v.at[0], out_hbm.at[pl.ds(out0 + ch0 * _CH * _D, _CH * _D)], sem0)
        ch1 = 2 * p + 1
        fill(1, ch1)
        c1 = pltpu.async_copy(
            stage_v.at[1], out_hbm.at[pl.ds(out0 + ch1 * _CH * _D, _CH * _D)], sem1)
        c0.wait()
        c1.wait()
        return carry

    lax.fori_loop(0, _NCH // 2, pair_body, 0)


def kernel(inputs, timestamp, train, W, b):
    del inputs, train
    table = (W.T + b[None, :]).reshape(_NBINS * _D)
    mesh = plsc.VectorSubcoreMesh(core_axis_name="c", subcore_axis_name="s")
    k = pl.kernel(
        _sc_body,
        out_type=jax.ShapeDtypeStruct((_B * _L * _D,), jnp.float32),
        mesh=mesh,
        compiler_params=pltpu.CompilerParams(use_tc_tiling_on_sc=False),
        scratch_types=[
            pltpu.VMEM((_RW, _L + 1), jnp.float32),
            pltpu.VMEM((_PW,), jnp.int32),
            pltpu.VMEM((_NBINS * _D,), jnp.float32),
            pltpu.VMEM((2, _CH * _D), jnp.float32),
            pltpu.SemaphoreType.DMA,
            pltpu.SemaphoreType.DMA,
        ],
    )
    out = k(timestamp, table)
    return (out.reshape(_B, _L, _D), timestamp[:, :-1])


# speedup vs baseline: 18.6355x; 1.3608x over previous
import jax
import jax.numpy as jnp
from jax import lax
from jax.experimental import pallas as pl
from jax.experimental.pallas import tpu as pltpu
from jax.experimental.pallas import tpu_sc as plsc

_B, _L = 4096, 200
_NBINS = 101
_D = 64
_NC, _NS = 2, 16
_NW = _NC * _NS
_RW = _B // _NW

_OFFS = tuple(range(0, _L - 16, 16)) + (_L - 16,)


def _sc_body(ts_hbm, table_hbm, out_hbm, ts_v, idx_v, tab_v, stage_v, sem0, sem1):
    wid = lax.axis_index("s") * _NC + lax.axis_index("c")
    row0 = wid * _RW
    pltpu.sync_copy(table_hbm, tab_v)
    pltpu.sync_copy(ts_hbm.at[pl.ds(row0 * (_L + 1), _RW * (_L + 1))], ts_v)

    def row_body(r, carry):
        tsb = r * (_L + 1)
        p0 = r * _L
        for off in _OFFS:
            a = ts_v[pl.ds(tsb + off + 1, 16)]
            c = ts_v[pl.ds(tsb + off, 16)]
            v = (a - c) * jnp.float32(_NBINS - 1)
            i = jnp.minimum(jnp.maximum(v.astype(jnp.int32), 0), _NBINS - 1)
            idx_v[pl.ds(p0 + off, 16)] = i * _D
        return carry

    lax.fori_loop(0, _RW, row_body, 0)

    def fill(buf, ch):
        pend = []
        for off in _OFFS:
            iv = idx_v[pl.ds(ch * _L + off, 16)]
            for r in range(16):
                base = iv[r]
                vals = [tab_v[pl.ds(base + c, 16)] for c in range(0, _D, 16)]
                if pend:
                    prow, pvals = pend.pop()
                    for k in range(_D // 16):
                        stage_v[buf, prow, pl.ds(k * 16, 16)] = pvals[k]
                pend.append((off + r, vals))
        prow, pvals = pend.pop()
        for k in range(_D // 16):
            stage_v[buf, prow, pl.ds(k * 16, 16)] = pvals[k]

    def pair_body(p, carry):
        ch0 = 2 * p
        fill(0, ch0)
        c0 = pltpu.async_copy(stage_v.at[0], out_hbm.at[row0 + ch0], sem0)
        ch1 = 2 * p + 1
        fill(1, ch1)
        c1 = pltpu.async_copy(stage_v.at[1], out_hbm.at[row0 + ch1], sem1)
        c0.wait()
        c1.wait()
        return carry

    lax.fori_loop(0, _RW // 2, pair_body, 0)


def kernel(inputs, timestamp, train, W, b):
    del inputs, train
    table = (W.T + b[None, :]).reshape(_NBINS * _D)
    mesh = plsc.VectorSubcoreMesh(core_axis_name="c", subcore_axis_name="s")
    k = pl.kernel(
        _sc_body,
        out_type=jax.ShapeDtypeStruct((_B, _L, _D), jnp.float32),
        mesh=mesh,
        scratch_types=[
            pltpu.VMEM((_RW * (_L + 1),), jnp.float32),
            pltpu.VMEM((_RW * _L,), jnp.int32),
            pltpu.VMEM((_NBINS * _D,), jnp.float32),
            pltpu.VMEM((2, _L, _D), jnp.float32),
            pltpu.SemaphoreType.DMA,
            pltpu.SemaphoreType.DMA,
        ],
    )
    out = k(timestamp.reshape(_B * (_L + 1)), table)
    return (out, timestamp[:, :-1])


# speedup vs baseline: 24.2584x; 1.3017x over previous
import jax
import jax.numpy as jnp
from jax import lax
from jax.experimental import pallas as pl
from jax.experimental.pallas import tpu as pltpu
from jax.experimental.pallas import tpu_sc as plsc

_B, _L = 4096, 200
_NBINS = 101
_D = 64
_TS = 65
_NC, _NS = 2, 16
_LG = 8
_BQ = 4
_LW = _L // _LG
_BW = _B // _BQ


def _sc_body(ts_hbm, table_hbm, out_hbm, ts_v, idx_v, tab_v, stage_v, sem0, sem1, semi):
    wid = lax.axis_index("s") * _NC + lax.axis_index("c")
    lg = wid // _BQ
    bq = wid - lg * _BQ
    l0 = lg * _LW
    b0 = bq * _BW

    cps = [pltpu.async_copy(
        ts_hbm.at[pl.ds((l0 + ll) * _B + b0, _BW)],
        ts_v.at[pl.ds(ll * _BW, _BW)], semi) for ll in range(_LW + 1)]
    pltpu.sync_copy(table_hbm, tab_v)
    for cp in cps:
        cp.wait()

    def l_body(ll, carry):
        for v in range(_BW // 16):
            cur = ts_v[pl.ds(ll * _BW + v * 16, 16)]
            nxt = ts_v[pl.ds((ll + 1) * _BW + v * 16, 16)]
            d = (nxt - cur) * jnp.float32(_NBINS - 1)
            i = jnp.minimum(jnp.maximum(d.astype(jnp.int32), 0), _NBINS - 1)
            idx_v[pl.ds(ll * _BW + v * 16, 16)] = i * _TS
        return carry

    lax.fori_loop(0, _LW, l_body, 0)

    def fill(buf, ll, j):
        def b_body(b16, carry):
            iv = idx_v[pl.ds(ll * _BW + j * 128 + b16 * 16, 16)]
            for d in range(_D):
                g = plsc.load_gather(tab_v, [iv + d])
                stage_v[buf, d // 8, d % 8, pl.ds(b16 * 16, 16)] = g
            return carry
        lax.fori_loop(0, 8, b_body, 0)

    def pair_body(p, carry):
        ll = p // (_BQ)
        jp = p - ll * _BQ
        j0 = 2 * jp
        fill(0, ll, j0)
        c0 = pltpu.async_copy(
            stage_v.at[0], out_hbm.at[l0 + ll, :, 8 * bq + j0], sem0)
        j1 = j0 + 1
        fill(1, ll, j1)
        c1 = pltpu.async_copy(
            stage_v.at[1], out_hbm.at[l0 + ll, :, 8 * bq + j1], sem1)
        c0.wait()
        c1.wait()
        return carry

    lax.fori_loop(0, _LW * _BQ, pair_body, 0)


def kernel(inputs, timestamp, train, W, b):
    del inputs, train
    table = jnp.pad(W.T + b[None, :], ((0, 0), (0, 1))).reshape(_NBINS * _TS)
    mesh = plsc.VectorSubcoreMesh(core_axis_name="c", subcore_axis_name="s")
    k = pl.kernel(
        _sc_body,
        out_type=jax.ShapeDtypeStruct((_L, _D // 8, _B // 128, 8, 128), jnp.float32),
        mesh=mesh,
        compiler_params=pltpu.CompilerParams(
            use_tc_tiling_on_sc=False, needs_layout_passes=False),
        scratch_types=[
            pltpu.VMEM(((_LW + 1) * _BW,), jnp.float32),
            pltpu.VMEM((_LW * _BW,), jnp.int32),
            pltpu.VMEM((_NBINS * _TS,), jnp.float32),
            pltpu.VMEM((2, 8, 8, 128), jnp.float32),
            pltpu.SemaphoreType.DMA,
            pltpu.SemaphoreType.DMA,
            pltpu.SemaphoreType.DMA,
        ],
    )
    out5 = k(timestamp.T.reshape(_B * (_L + 1)), table)
    out = out5.transpose(2, 4, 0, 1, 3).reshape(_B, _L, _D)
    return (out, timestamp[:, :-1])
